# baseline (speedup 1.0000x reference)
import functools

import jax
import jax.numpy as jnp
import numpy as np
from jax import lax
from jax.experimental import pallas as pl
from jax.experimental.pallas import tpu as pltpu
from jax.experimental.pallas import tpu_sc as plsc

B, N, D, FH, FW, C = 2, 6, 41, 16, 44, 64
OGH, OGW = 256, 704
NX, NY, NZ = 256, 256, 1
DX = np.array([0.4, 0.4, 20.0], dtype=np.float32)
BXv = np.array([-51.0, -51.0, 0.0], dtype=np.float32)

NPRIME = B * N * D * FH * FW
PER_B = NPRIME // B
NSUB = 16
PER_TILE = PER_B // NSUB
STAGE = PER_TILE + 8
SCAN_BLOCKS = STAGE // 16
LIST_CAP = 12288
PUB = 2048
CHUNK_ROWS = 8192
CHUNKS_PER_CORE = 8
ROWS_PER_TILE = CHUNK_ROWS // NSUB
TRASH_ROW = CHUNK_ROWS
GRID_ROWS = B * NY * NX
SENTINEL = np.int32(1 << 17)
GBLK = 128
NSLOT = 6
DEPTH = 4


def _frustum():
    ds = np.broadcast_to((4.0 + np.arange(D, dtype=np.float32)).reshape(D, 1, 1), (D, FH, FW))
    xs = np.broadcast_to(np.linspace(0, OGW - 1, FW, dtype=np.float32).reshape(1, 1, FW), (D, FH, FW))
    ys = np.broadcast_to(np.linspace(0, OGH - 1, FH, dtype=np.float32).reshape(1, FH, 1), (D, FH, FW))
    return jnp.asarray(np.stack([xs, ys, ds], axis=-1))


def _point_rows(rots, trans, intrins, post_rots, post_trans):
    frustum = _frustum()
    points = frustum[None, None] - post_trans[:, :, None, None, None, :]
    points = jnp.matmul(jnp.linalg.inv(post_rots)[:, :, None, None, None], points[..., None])
    points = jnp.concatenate([points[..., :2, :] * points[..., 2:3, :], points[..., 2:3, :]], axis=-2)
    combine = jnp.matmul(rots, jnp.linalg.inv(intrins))
    points = jnp.matmul(combine[:, :, None, None, None], points)[..., 0]
    points = points + trans[:, :, None, None, None, :]
    geom = ((points - jnp.asarray(BXv - DX / 2.0)) / jnp.asarray(DX)).astype(jnp.int32).reshape(NPRIME, 3)
    kept = ((geom[:, 0] >= 0) & (geom[:, 0] < NX) & (geom[:, 1] >= 0) & (geom[:, 1] < NY)
            & (geom[:, 2] >= 0) & (geom[:, 2] < NZ))
    batch_ix = jnp.repeat(jnp.arange(B, dtype=jnp.int32), PER_B)
    rows = batch_ix * (NY * NX) + geom[:, 1] * NX + geom[:, 0]
    return jnp.where(kept, rows, SENTINEL)


@functools.partial(
    pl.kernel,
    out_type=jax.ShapeDtypeStruct((GRID_ROWS, C), jnp.float32),
    mesh=plsc.VectorSubcoreMesh(core_axis_name="c", subcore_axis_name="s"),
    scratch_types=[
        pltpu.VMEM((STAGE,), jnp.int32),
        pltpu.VMEM((LIST_CAP,), jnp.int32),
        pltpu.VMEM((NSLOT, GBLK), jnp.int32),
        pltpu.VMEM((NSLOT, GBLK), jnp.int32),
        pltpu.VMEM((NSLOT, GBLK), jnp.int32),
        pltpu.VMEM((NSLOT, GBLK, C), jnp.float32),
        pltpu.VMEM((128, C), jnp.float32),
        pltpu.VMEM((16, 16), jnp.int32),
        pltpu.VMEM((16,), jnp.int32),
        pltpu.VMEM_SHARED((CHUNK_ROWS + 8, C), jnp.float32),
        pltpu.VMEM_SHARED((NSUB, LIST_CAP), jnp.int32),
        pltpu.VMEM_SHARED((NSUB, 16), jnp.int32),
        pltpu.SemaphoreType.DMA((NSLOT,)),
        pltpu.SemaphoreType.DMA((NSLOT,)),
        pltpu.SemaphoreType.DMA((4,)),
    ],
    compiler_params=pltpu.CompilerParams(needs_layout_passes=False,
                                         use_tc_tiling_on_sc=False),
)
def _pool_kernel(rows_hbm, xf_hbm, out_hbm,
                 stage, plist, pblk, idxblk, lrowblk, gbuf, zbuf,
                 cntbuf, nbv, slab, spub, scnt, gsem, ssem, zsem):
    c = lax.axis_index("c")
    s = lax.axis_index("s")
    zero16 = jnp.zeros((16,), jnp.float32)

    @pl.loop(0, 128)
    def _zb(r):
        for k in range(C // 16):
            zbuf[r, pl.ds(k * 16, 16)] = zero16

    base_pt = c * PER_B + s * PER_TILE
    pltpu.sync_copy(rows_hbm.at[pl.ds(base_pt, STAGE)], stage)

    for chunk in range(CHUNKS_PER_CORE):
        target = c * CHUNKS_PER_CORE + chunk
        row_base = target * CHUNK_ROWS

        for k in range(ROWS_PER_TILE // 128):
            pltpu.async_copy(zbuf, slab.at[pl.ds(s * ROWS_PER_TILE + k * 128, 128)],
                             zsem.at[k])

        def scan_body(j, cnt):
            r = stage[pl.ds(j * 16, 16)]
            lane = j * 16 + lax.iota(jnp.int32, 16)
            m = (lax.shift_right_logical(r, 13) == target) & (lane < PER_TILE)
            pidx_local = s * PER_TILE + lane
            packed = lax.shift_left(r - row_base, 18) | pidx_local
            mi = plsc.cumsum(m.astype(jnp.int32))
            dest = jnp.where(m, cnt + mi - 1, 0)
            plsc.store_scatter(plist, [dest], packed, mask=m)
            return cnt + jnp.sum(m.astype(jnp.int32))

        cnt = lax.fori_loop(0, SCAN_BLOCKS, scan_body, jnp.int32(0), unroll=8)

        dummy = jnp.zeros((16,), jnp.int32) + np.int32(-(1 << 31))
        for k in range(GBLK // 16):
            plist[pl.ds(cnt + k * 16, 16)] = dummy
        nb = (cnt + (GBLK - 1)) // GBLK

        npg = (nb * GBLK + (PUB - 1)) // PUB

        def pub_body(g, carry):
            pltpu.sync_copy(plist.at[pl.ds(g * PUB, PUB)],
                            spub.at[s, pl.ds(g * PUB, PUB)])
            return carry

        lax.fori_loop(0, npg, pub_body, jnp.int32(0))
        nbv[pl.ds(0, 16)] = jnp.zeros((16,), jnp.int32) + nb
        pltpu.sync_copy(nbv, scnt.at[s])
        for k in range(ROWS_PER_TILE // 128):
            pltpu.make_async_copy(
                zbuf, slab.at[pl.ds(s * ROWS_PER_TILE + k * 128, 128)],
                zsem.at[k]).wait()
        plsc.subcore_barrier()

        pltpu.sync_copy(scnt, cntbuf)

        def sum_body(t, acc):
            return acc + cntbuf[t, pl.ds(0, 16)][0]

        nbtot = lax.fori_loop(0, NSUB, sum_body, jnp.int32(0))
        nmine = jnp.maximum(nbtot - s + (NSUB - 1), 0) // NSUB

        def fetch_block(i, slot):
            j = s + i * NSUB

            def find(t, carry):
                acc, owner, lb = carry
                nbt = cntbuf[t, pl.ds(0, 16)][0]
                hit = (j >= acc) & (j < acc + nbt)
                return (acc + nbt,
                        jnp.where(hit, t, owner),
                        jnp.where(hit, j - acc, lb))

            _, o, lb = lax.fori_loop(0, NSUB, find,
                                     (jnp.int32(0), jnp.int32(0), jnp.int32(0)))
            pltpu.sync_copy(spub.at[o, pl.ds(lb * GBLK, GBLK)], pblk.at[slot])
            for k in range(GBLK // 16):
                v = pblk[slot, pl.ds(k * 16, 16)]
                lrowblk[slot, pl.ds(k * 16, 16)] = lax.shift_right_logical(v, 18)
                idxblk[slot, pl.ds(k * 16, 16)] = (v & 0x3FFFF) + c * PER_B
            pltpu.async_copy(xf_hbm.at[idxblk.at[slot]], gbuf.at[slot],
                             gsem.at[slot])

        def scatter_desc(slot):
            return pltpu.make_async_copy(gbuf.at[slot], slab.at[lrowblk.at[slot]],
                                         ssem.at[slot])

        for i in range(DEPTH):
            @pl.when(i < nmine)
            def _prime():
                fetch_block(jnp.int32(i), jnp.int32(i))

        def gs_body(t, carry):
            slot = lax.rem(t, NSLOT)
            nxt = t + DEPTH

            @pl.when(nxt < nmine)
            def _prefetch():
                nslot = lax.rem(nxt, NSLOT)

                @pl.when(nxt >= NSLOT)
                def _drain():
                    scatter_desc(nslot).wait()
                fetch_block(nxt, nslot)

            pltpu.make_async_copy(xf_hbm.at[idxblk.at[slot]], gbuf.at[slot],
                                  gsem.at[slot]).wait()
            pltpu.async_copy(gbuf.at[slot], slab.at[lrowblk.at[slot]],
                             ssem.at[slot], add=True)
            return carry

        lax.fori_loop(0, nmine, gs_body, jnp.int32(0))
        for bb in range(NSLOT):
            @pl.when(bb < nmine)
            def _final_drain():
                scatter_desc(jnp.int32(bb)).wait()
        plsc.subcore_barrier()

        out_base = target * CHUNK_ROWS + s * ROWS_PER_TILE
        pltpu.sync_copy(slab.at[pl.ds(s * ROWS_PER_TILE, ROWS_PER_TILE)],
                        out_hbm.at[pl.ds(out_base, ROWS_PER_TILE)])
        plsc.subcore_barrier()


def kernel(x, rots, trans, intrins, post_rots, post_trans):
    rows = _point_rows(rots, trans, intrins, post_rots, post_trans)
    rows = jnp.concatenate([rows, jnp.full((32,), SENTINEL, jnp.int32)])
    xf = x.reshape(NPRIME, C)
    grid = _pool_kernel(rows, xf)
    return grid.reshape(B, NY, NX, C).transpose(0, 3, 1, 2)


# speedup vs baseline: 1.0041x; 1.0041x over previous
import functools

import jax
import jax.numpy as jnp
import numpy as np
from jax import lax
from jax.experimental import pallas as pl
from jax.experimental.pallas import tpu as pltpu
from jax.experimental.pallas import tpu_sc as plsc

B, N, D, FH, FW, C = 2, 6, 41, 16, 44, 64
OGH, OGW = 256, 704
NX, NY, NZ = 256, 256, 1
DX = np.array([0.4, 0.4, 20.0], dtype=np.float32)
BXv = np.array([-51.0, -51.0, 0.0], dtype=np.float32)

NPRIME = B * N * D * FH * FW
PER_B = NPRIME // B
NSUB = 16
PER_TILE = PER_B // NSUB
STAGE = PER_TILE + 8
SCAN_BLOCKS = STAGE // 16
LIST_CAP = 12288
PUB = 2048
CHUNK_ROWS = 8192
CHUNKS_PER_CORE = 8
ROWS_PER_TILE = CHUNK_ROWS // NSUB
TRASH_ROW = CHUNK_ROWS
GRID_ROWS = B * NY * NX
SENTINEL = np.int32(1 << 17)
GBLK = 128
NSLOT = 6
DEPTH = 4


def _frustum():
    ds = np.broadcast_to((4.0 + np.arange(D, dtype=np.float32)).reshape(D, 1, 1), (D, FH, FW))
    xs = np.broadcast_to(np.linspace(0, OGW - 1, FW, dtype=np.float32).reshape(1, 1, FW), (D, FH, FW))
    ys = np.broadcast_to(np.linspace(0, OGH - 1, FH, dtype=np.float32).reshape(1, FH, 1), (D, FH, FW))
    return jnp.asarray(np.stack([xs, ys, ds], axis=-1))


def _point_rows(rots, trans, intrins, post_rots, post_trans):
    frustum = _frustum()
    points = frustum[None, None] - post_trans[:, :, None, None, None, :]
    points = jnp.matmul(jnp.linalg.inv(post_rots)[:, :, None, None, None], points[..., None])
    points = jnp.concatenate([points[..., :2, :] * points[..., 2:3, :], points[..., 2:3, :]], axis=-2)
    combine = jnp.matmul(rots, jnp.linalg.inv(intrins))
    points = jnp.matmul(combine[:, :, None, None, None], points)[..., 0]
    points = points + trans[:, :, None, None, None, :]
    geom = ((points - jnp.asarray(BXv - DX / 2.0)) / jnp.asarray(DX)).astype(jnp.int32).reshape(NPRIME, 3)
    kept = ((geom[:, 0] >= 0) & (geom[:, 0] < NX) & (geom[:, 1] >= 0) & (geom[:, 1] < NY)
            & (geom[:, 2] >= 0) & (geom[:, 2] < NZ))
    batch_ix = jnp.repeat(jnp.arange(B, dtype=jnp.int32), PER_B)
    rows = batch_ix * (NY * NX) + geom[:, 1] * NX + geom[:, 0]
    return jnp.where(kept, rows, SENTINEL)


@functools.partial(
    pl.kernel,
    out_type=jax.ShapeDtypeStruct((GRID_ROWS, C), jnp.float32),
    mesh=plsc.VectorSubcoreMesh(core_axis_name="c", subcore_axis_name="s"),
    scratch_types=[
        pltpu.VMEM((STAGE,), jnp.int32),
        pltpu.VMEM((LIST_CAP,), jnp.int32),
        pltpu.VMEM((NSLOT, GBLK), jnp.int32),
        pltpu.VMEM((NSLOT, GBLK), jnp.int32),
        pltpu.VMEM((NSLOT, GBLK), jnp.int32),
        pltpu.VMEM((NSLOT, GBLK, C), jnp.float32),
        pltpu.VMEM((128, C), jnp.float32),
        pltpu.VMEM((16, 16), jnp.int32),
        pltpu.VMEM((16,), jnp.int32),
        pltpu.VMEM_SHARED((CHUNK_ROWS + 8, C), jnp.float32),
        pltpu.VMEM_SHARED((NSUB, LIST_CAP), jnp.int32),
        pltpu.VMEM_SHARED((NSUB, 16), jnp.int32),
        pltpu.SemaphoreType.DMA((NSLOT,)),
        pltpu.SemaphoreType.DMA((NSLOT,)),
        pltpu.SemaphoreType.DMA((NSLOT,)),
        pltpu.SemaphoreType.DMA((4,)),
    ],
    compiler_params=pltpu.CompilerParams(needs_layout_passes=False,
                                         use_tc_tiling_on_sc=False),
)
def _pool_kernel(rows_hbm, xf_hbm, out_hbm,
                 stage, plist, pblk, idxblk, lrowblk, gbuf, zbuf,
                 cntbuf, nbv, slab, spub, scnt, gsem, ssem, psem, zsem):
    c = lax.axis_index("c")
    s = lax.axis_index("s")
    zero16 = jnp.zeros((16,), jnp.float32)

    @pl.loop(0, 128)
    def _zb(r):
        for k in range(C // 16):
            zbuf[r, pl.ds(k * 16, 16)] = zero16

    base_pt = c * PER_B + s * PER_TILE
    pltpu.sync_copy(rows_hbm.at[pl.ds(base_pt, STAGE)], stage)

    for chunk in range(CHUNKS_PER_CORE):
        target = c * CHUNKS_PER_CORE + chunk
        row_base = target * CHUNK_ROWS

        for k in range(ROWS_PER_TILE // 128):
            pltpu.async_copy(zbuf, slab.at[pl.ds(s * ROWS_PER_TILE + k * 128, 128)],
                             zsem.at[k])

        def scan_body(j, cnt):
            r = stage[pl.ds(j * 16, 16)]
            lane = j * 16 + lax.iota(jnp.int32, 16)
            m = (lax.shift_right_logical(r, 13) == target) & (lane < PER_TILE)
            pidx_local = s * PER_TILE + lane
            packed = lax.shift_left(r - row_base, 18) | pidx_local
            mi = plsc.cumsum(m.astype(jnp.int32))
            dest = jnp.where(m, cnt + mi - 1, 0)
            plsc.store_scatter(plist, [dest], packed, mask=m)
            return cnt + jnp.sum(m.astype(jnp.int32))

        cnt = lax.fori_loop(0, SCAN_BLOCKS, scan_body, jnp.int32(0), unroll=4)

        dummy = jnp.zeros((16,), jnp.int32) + np.int32(-(1 << 31))
        for k in range(GBLK // 16):
            plist[pl.ds(cnt + k * 16, 16)] = dummy
        nb = (cnt + (GBLK - 1)) // GBLK

        npg = (nb * GBLK + (PUB - 1)) // PUB

        def pub_body(g, carry):
            pltpu.sync_copy(plist.at[pl.ds(g * PUB, PUB)],
                            spub.at[s, pl.ds(g * PUB, PUB)])
            return carry

        lax.fori_loop(0, npg, pub_body, jnp.int32(0))
        nbv[pl.ds(0, 16)] = jnp.zeros((16,), jnp.int32) + nb
        pltpu.sync_copy(nbv, scnt.at[s])
        for k in range(ROWS_PER_TILE // 128):
            pltpu.make_async_copy(
                zbuf, slab.at[pl.ds(s * ROWS_PER_TILE + k * 128, 128)],
                zsem.at[k]).wait()
        plsc.subcore_barrier()

        pltpu.sync_copy(scnt, cntbuf)

        def sum_body(t, acc):
            return acc + cntbuf[t, pl.ds(0, 16)][0]

        nbtot = lax.fori_loop(0, NSUB, sum_body, jnp.int32(0))
        nmine = jnp.maximum(nbtot - s + (NSUB - 1), 0) // NSUB

        def locate(i):
            j = s + i * NSUB

            def find(t, carry):
                acc, owner, lb = carry
                nbt = cntbuf[t, pl.ds(0, 16)][0]
                hit = (j >= acc) & (j < acc + nbt)
                return (acc + nbt,
                        jnp.where(hit, t, owner),
                        jnp.where(hit, j - acc, lb))

            _, o, lb = lax.fori_loop(0, NSUB, find,
                                     (jnp.int32(0), jnp.int32(0), jnp.int32(0)))
            return o, lb

        def pfetch_desc(i, slot):
            o, lb = locate(i)
            return pltpu.make_async_copy(spub.at[o, pl.ds(lb * GBLK, GBLK)],
                                         pblk.at[slot], psem.at[slot])

        def unpack_and_gather(slot):
            for k in range(GBLK // 16):
                v = pblk[slot, pl.ds(k * 16, 16)]
                lrowblk[slot, pl.ds(k * 16, 16)] = lax.shift_right_logical(v, 18)
                idxblk[slot, pl.ds(k * 16, 16)] = (v & 0x3FFFF) + c * PER_B
            pltpu.async_copy(xf_hbm.at[idxblk.at[slot]], gbuf.at[slot],
                             gsem.at[slot])

        def fetch_block(i, slot):
            pfetch_desc(i, slot).start()
            pfetch_desc(i, slot).wait()
            unpack_and_gather(slot)

        def scatter_desc(slot):
            return pltpu.make_async_copy(gbuf.at[slot], slab.at[lrowblk.at[slot]],
                                         ssem.at[slot])

        for i in range(DEPTH):
            @pl.when(i < nmine)
            def _prime():
                fetch_block(jnp.int32(i), jnp.int32(i))

        def gs_body(t, carry):
            slot = lax.rem(t, NSLOT)
            nxt = t + DEPTH

            @pl.when(nxt < nmine)
            def _prefetch():
                nslot = lax.rem(nxt, NSLOT)

                @pl.when(nxt >= NSLOT)
                def _drain():
                    scatter_desc(nslot).wait()
                pfetch_desc(nxt, nslot).start()

            prv = t + DEPTH - 1

            @pl.when((prv >= DEPTH) & (prv < nmine))
            def _unpack():
                uslot = lax.rem(prv, NSLOT)
                pltpu.make_async_copy(spub.at[0, pl.ds(0, GBLK)], pblk.at[uslot],
                                      psem.at[uslot]).wait()
                unpack_and_gather(uslot)

            pltpu.make_async_copy(xf_hbm.at[idxblk.at[slot]], gbuf.at[slot],
                                  gsem.at[slot]).wait()
            pltpu.async_copy(gbuf.at[slot], slab.at[lrowblk.at[slot]],
                             ssem.at[slot], add=True)
            return carry

        lax.fori_loop(0, nmine, gs_body, jnp.int32(0))
        for bb in range(NSLOT):
            @pl.when(bb < nmine)
            def _final_drain():
                scatter_desc(jnp.int32(bb)).wait()
        plsc.subcore_barrier()

        out_base = target * CHUNK_ROWS + s * ROWS_PER_TILE
        pltpu.sync_copy(slab.at[pl.ds(s * ROWS_PER_TILE, ROWS_PER_TILE)],
                        out_hbm.at[pl.ds(out_base, ROWS_PER_TILE)])
        plsc.subcore_barrier()


def kernel(x, rots, trans, intrins, post_rots, post_trans):
    rows = _point_rows(rots, trans, intrins, post_rots, post_trans)
    rows = jnp.concatenate([rows, jnp.full((32,), SENTINEL, jnp.int32)])
    xf = x.reshape(NPRIME, C)
    grid = _pool_kernel(rows, xf)
    return grid.reshape(B, NY, NX, C).transpose(0, 3, 1, 2)


# speedup vs baseline: 1.0070x; 1.0029x over previous
import functools

import jax
import jax.numpy as jnp
import numpy as np
from jax import lax
from jax.experimental import pallas as pl
from jax.experimental.pallas import tpu as pltpu
from jax.experimental.pallas import tpu_sc as plsc

B, N, D, FH, FW, C = 2, 6, 41, 16, 44, 64
OGH, OGW = 256, 704
NX, NY, NZ = 256, 256, 1
DX = np.array([0.4, 0.4, 20.0], dtype=np.float32)
BXv = np.array([-51.0, -51.0, 0.0], dtype=np.float32)

NPRIME = B * N * D * FH * FW
PER_B = NPRIME // B
NSUB = 16
PER_TILE = PER_B // NSUB
STAGE = PER_TILE + 8
SCAN_BLOCKS = STAGE // 16
LIST_CAP = 12288
PUB = 2048
CHUNK_ROWS = 8192
CHUNKS_PER_CORE = 8
ROWS_PER_TILE = CHUNK_ROWS // NSUB
TRASH_ROW = CHUNK_ROWS
GRID_ROWS = B * NY * NX
SENTINEL = np.int32(1 << 17)
GBLK = 128
NSLOT = 6
DEPTH = 4


def _frustum():
    ds = np.broadcast_to((4.0 + np.arange(D, dtype=np.float32)).reshape(D, 1, 1), (D, FH, FW))
    xs = np.broadcast_to(np.linspace(0, OGW - 1, FW, dtype=np.float32).reshape(1, 1, FW), (D, FH, FW))
    ys = np.broadcast_to(np.linspace(0, OGH - 1, FH, dtype=np.float32).reshape(1, FH, 1), (D, FH, FW))
    return jnp.asarray(np.stack([xs, ys, ds], axis=-1))


def _point_rows(rots, trans, intrins, post_rots, post_trans):
    frustum = _frustum()
    points = frustum[None, None] - post_trans[:, :, None, None, None, :]
    points = jnp.matmul(jnp.linalg.inv(post_rots)[:, :, None, None, None], points[..., None])
    points = jnp.concatenate([points[..., :2, :] * points[..., 2:3, :], points[..., 2:3, :]], axis=-2)
    combine = jnp.matmul(rots, jnp.linalg.inv(intrins))
    points = jnp.matmul(combine[:, :, None, None, None], points)[..., 0]
    points = points + trans[:, :, None, None, None, :]
    geom = ((points - jnp.asarray(BXv - DX / 2.0)) / jnp.asarray(DX)).astype(jnp.int32).reshape(NPRIME, 3)
    kept = ((geom[:, 0] >= 0) & (geom[:, 0] < NX) & (geom[:, 1] >= 0) & (geom[:, 1] < NY)
            & (geom[:, 2] >= 0) & (geom[:, 2] < NZ))
    batch_ix = jnp.repeat(jnp.arange(B, dtype=jnp.int32), PER_B)
    rows = batch_ix * (NY * NX) + geom[:, 1] * NX + geom[:, 0]
    return jnp.where(kept, rows, SENTINEL)


@functools.partial(
    pl.kernel,
    out_type=jax.ShapeDtypeStruct((GRID_ROWS, C), jnp.float32),
    mesh=plsc.VectorSubcoreMesh(core_axis_name="c", subcore_axis_name="s"),
    scratch_types=[
        pltpu.VMEM((STAGE,), jnp.int32),
        pltpu.VMEM((LIST_CAP,), jnp.int32),
        pltpu.VMEM((NSLOT, GBLK), jnp.int32),
        pltpu.VMEM((NSLOT, GBLK), jnp.int32),
        pltpu.VMEM((NSLOT, GBLK), jnp.int32),
        pltpu.VMEM((NSLOT, GBLK, C), jnp.float32),
        pltpu.VMEM((128, C), jnp.float32),
        pltpu.VMEM((16, 16), jnp.int32),
        pltpu.VMEM((16,), jnp.int32),
        pltpu.VMEM_SHARED((CHUNK_ROWS + 8, C), jnp.float32),
        pltpu.VMEM_SHARED((NSUB, LIST_CAP), jnp.int32),
        pltpu.VMEM_SHARED((NSUB, 16), jnp.int32),
        pltpu.SemaphoreType.DMA((NSLOT,)),
        pltpu.SemaphoreType.DMA((NSLOT,)),
        pltpu.SemaphoreType.DMA((4,)),
    ],
    compiler_params=pltpu.CompilerParams(needs_layout_passes=False,
                                         use_tc_tiling_on_sc=False),
)
def _pool_kernel(rows_hbm, xf_hbm, out_hbm,
                 stage, plist, pblk, idxblk, lrowblk, gbuf, zbuf,
                 cntbuf, nbv, slab, spub, scnt, gsem, ssem, zsem):
    c = lax.axis_index("c")
    s = lax.axis_index("s")
    zero16 = jnp.zeros((16,), jnp.float32)

    @pl.loop(0, 128)
    def _zb(r):
        for k in range(C // 16):
            zbuf[r, pl.ds(k * 16, 16)] = zero16

    base_pt = c * PER_B + s * PER_TILE
    pltpu.sync_copy(rows_hbm.at[pl.ds(base_pt, STAGE)], stage)

    for chunk in range(CHUNKS_PER_CORE):
        target = c * CHUNKS_PER_CORE + chunk
        row_base = target * CHUNK_ROWS

        for k in range(ROWS_PER_TILE // 128):
            pltpu.async_copy(zbuf, slab.at[pl.ds(s * ROWS_PER_TILE + k * 128, 128)],
                             zsem.at[k])

        def scan_body(j, cnt):
            r = stage[pl.ds(j * 16, 16)]
            lane = j * 16 + lax.iota(jnp.int32, 16)
            m = (lax.shift_right_logical(r, 13) == target) & (lane < PER_TILE)
            pidx_local = s * PER_TILE + lane
            packed = lax.shift_left(r - row_base, 18) | pidx_local
            mi = plsc.cumsum(m.astype(jnp.int32))
            dest = jnp.where(m, cnt + mi - 1, 0)
            plsc.store_scatter(plist, [dest], packed, mask=m)
            return cnt + jnp.sum(m.astype(jnp.int32))

        cnt = lax.fori_loop(0, SCAN_BLOCKS, scan_body, jnp.int32(0), unroll=4)

        dummy = jnp.zeros((16,), jnp.int32) + np.int32(-(1 << 31))
        for k in range(GBLK // 16):
            plist[pl.ds(cnt + k * 16, 16)] = dummy
        nb = (cnt + (GBLK - 1)) // GBLK

        npg = (nb * GBLK + (PUB - 1)) // PUB

        def pub_body(g, carry):
            pltpu.sync_copy(plist.at[pl.ds(g * PUB, PUB)],
                            spub.at[s, pl.ds(g * PUB, PUB)])
            return carry

        lax.fori_loop(0, npg, pub_body, jnp.int32(0))
        nbv[pl.ds(0, 16)] = jnp.zeros((16,), jnp.int32) + nb
        pltpu.sync_copy(nbv, scnt.at[s])
        for k in range(ROWS_PER_TILE // 128):
            pltpu.make_async_copy(
                zbuf, slab.at[pl.ds(s * ROWS_PER_TILE + k * 128, 128)],
                zsem.at[k]).wait()
        plsc.subcore_barrier()

        pltpu.sync_copy(scnt, cntbuf)

        def sum_body(t, acc):
            return acc + cntbuf[t, pl.ds(0, 16)][0]

        nbtot = lax.fori_loop(0, NSUB, sum_body, jnp.int32(0))
        nmine = jnp.maximum(nbtot - s + (NSUB - 1), 0) // NSUB

        def fetch_block(i, slot):
            j = s + i * NSUB

            def find(t, carry):
                acc, owner, lb = carry
                nbt = cntbuf[t, pl.ds(0, 16)][0]
                hit = (j >= acc) & (j < acc + nbt)
                return (acc + nbt,
                        jnp.where(hit, t, owner),
                        jnp.where(hit, j - acc, lb))

            _, o, lb = lax.fori_loop(0, NSUB, find,
                                     (jnp.int32(0), jnp.int32(0), jnp.int32(0)))
            pltpu.sync_copy(spub.at[o, pl.ds(lb * GBLK, GBLK)], pblk.at[slot])
            for k in range(GBLK // 16):
                v = pblk[slot, pl.ds(k * 16, 16)]
                lrowblk[slot, pl.ds(k * 16, 16)] = lax.shift_right_logical(v, 18)
                idxblk[slot, pl.ds(k * 16, 16)] = (v & 0x3FFFF) + c * PER_B
            pltpu.async_copy(xf_hbm.at[idxblk.at[slot]], gbuf.at[slot],
                             gsem.at[slot])

        def scatter_desc(slot):
            return pltpu.make_async_copy(gbuf.at[slot], slab.at[lrowblk.at[slot]],
                                         ssem.at[slot])

        for i in range(DEPTH):
            @pl.when(i < nmine)
            def _prime():
                fetch_block(jnp.int32(i), jnp.int32(i))

        def gs_body(t, carry):
            slot = lax.rem(t, NSLOT)
            nxt = t + DEPTH

            @pl.when(nxt < nmine)
            def _prefetch():
                nslot = lax.rem(nxt, NSLOT)

                @pl.when(nxt >= NSLOT)
                def _drain():
                    scatter_desc(nslot).wait()
                fetch_block(nxt, nslot)

            pltpu.make_async_copy(xf_hbm.at[idxblk.at[slot]], gbuf.at[slot],
                                  gsem.at[slot]).wait()
            pltpu.async_copy(gbuf.at[slot], slab.at[lrowblk.at[slot]],
                             ssem.at[slot], add=True)
            return carry

        lax.fori_loop(0, nmine, gs_body, jnp.int32(0))
        for bb in range(NSLOT):
            @pl.when(bb < nmine)
            def _final_drain():
                scatter_desc(jnp.int32(bb)).wait()
        plsc.subcore_barrier()

        out_base = target * CHUNK_ROWS + s * ROWS_PER_TILE
        pltpu.sync_copy(slab.at[pl.ds(s * ROWS_PER_TILE, ROWS_PER_TILE)],
                        out_hbm.at[pl.ds(out_base, ROWS_PER_TILE)])
        plsc.subcore_barrier()


def kernel(x, rots, trans, intrins, post_rots, post_trans):
    rows = _point_rows(rots, trans, intrins, post_rots, post_trans)
    rows = jnp.concatenate([rows, jnp.full((32,), SENTINEL, jnp.int32)])
    xf = x.reshape(NPRIME, C)
    grid = _pool_kernel(rows, xf)
    return grid.reshape(B, NY, NX, C).transpose(0, 3, 1, 2)
